# Initial kernel scaffold; baseline (speedup 1.0000x reference)
import functools

import jax
import jax.numpy as jnp
from jax import lax
from jax.experimental import pallas as pl
from jax.experimental.pallas import tpu as pltpu, tpu_sc as plsc

N = 10000
E = 320000
IN_DIM = 128
HID = 256
OUT_DIM = 128

N_PAD = 10240
E_PAD = 327680
CH = 128
NCH = E_PAD // CH
N_PT = N_PAD // 16

_mesh = plsc.VectorSubcoreMesh(core_axis_name="c", subcore_axis_name="s")


@functools.partial(
    pl.kernel,
    out_type=jax.ShapeDtypeStruct((2, N_PAD), jnp.float32),
    mesh=_mesh,
    scratch_types=[
        pltpu.VMEM((NCH // 32, CH), jnp.int32),
        pltpu.VMEM((CH,), jnp.float32),
        pltpu.VMEM((N_PT,), jnp.float32),
        pltpu.VMEM_SHARED((N_PAD,), jnp.float32),
    ],
)
def _deg(dst_hbm, ones_hbm, out_hbm, dst_v, ones_v, buf_v, acc_sh):
    c = lax.axis_index("c")
    s = lax.axis_index("s")
    wid = c * 16 + s
    nch = NCH // 32

    def _z(i, carry):
        buf_v[pl.ds(i * 16, 16)] = jnp.zeros((16,), jnp.float32)
        return carry

    lax.fori_loop(0, N_PT // 16, _z, 0)
    pltpu.sync_copy(buf_v, acc_sh.at[pl.ds(s * N_PT, N_PT)])
    pltpu.sync_copy(ones_hbm, ones_v)
    pltpu.sync_copy(dst_hbm.at[pl.ds(wid * nch, nch)], dst_v)
    plsc.subcore_barrier()

    def _body(j, carry):
        pltpu.sync_copy(ones_v, acc_sh.at[dst_v.at[j]], add=True)
        return carry

    lax.fori_loop(0, nch, _body, 0)
    plsc.subcore_barrier()
    pltpu.sync_copy(acc_sh.at[pl.ds(s * N_PT, N_PT)], buf_v)
    pltpu.sync_copy(buf_v, out_hbm.at[c, pl.ds(s * N_PT, N_PT)])


def _agg_body(g_hbm, src_hbm, dst_hbm, zeros_hbm, out_hbm,
              src_v, dst_v, msg0, msg1, sem0, sem1, acc_sh,
              *, nch, core_split_edges):
    c = lax.axis_index("c")
    s = lax.axis_index("s")
    row0 = (c * 16 + s) * nch if core_split_edges else s * nch

    pltpu.sync_copy(zeros_hbm, msg0)
    for k in range(N_PT // CH):
        pltpu.sync_copy(msg0, acc_sh.at[pl.ds(s * N_PT + k * CH, CH)])
    if core_split_edges:
        pltpu.sync_copy(src_hbm.at[pl.ds(row0, nch)], src_v)
    else:
        pltpu.sync_copy(src_hbm.at[c, pl.ds(row0, nch)], src_v)
    pltpu.sync_copy(dst_hbm.at[pl.ds(row0, nch)], dst_v)
    plsc.subcore_barrier()

    bufs = (msg0, msg1)
    sems = (sem0, sem1)
    pltpu.async_copy(g_hbm.at[src_v.at[0]], msg0, sem0)
    pltpu.async_copy(g_hbm.at[src_v.at[1]], msg1, sem1)

    @functools.partial(pl.loop, 0, nch, step=2)
    def _pipe(jo):
        for b in range(2):
            j = jo + b
            buf, sem = bufs[b], sems[b]
            pltpu.make_async_copy(g_hbm.at[pl.ds(0, CH)], buf, sem).wait()
            pltpu.sync_copy(buf, acc_sh.at[dst_v.at[j]], add=True)
            nxt = j + 2

            @pl.when(nxt < nch)
            def _():
                pltpu.async_copy(g_hbm.at[src_v.at[nxt]], buf, sem)

    plsc.subcore_barrier()
    for k in range(N_PT // CH):
        r = s * N_PT + k * CH
        pltpu.sync_copy(acc_sh.at[pl.ds(r, CH)], msg0)
        pltpu.sync_copy(msg0, out_hbm.at[c, pl.ds(r, CH)])


@functools.partial(
    pl.kernel,
    out_type=jax.ShapeDtypeStruct((2, N_PAD, 128), jnp.float32),
    mesh=_mesh,
    scratch_types=[
        pltpu.VMEM((NCH // 16, CH), jnp.int32),
        pltpu.VMEM((NCH // 16, CH), jnp.int32),
        pltpu.VMEM((CH, 128), jnp.float32),
        pltpu.VMEM((CH, 128), jnp.float32),
        pltpu.SemaphoreType.DMA,
        pltpu.SemaphoreType.DMA,
        pltpu.VMEM_SHARED((N_PAD, 128), jnp.float32),
    ],
)
def _agg1(g_hbm, src_hbm, dst_hbm, zeros_hbm, out_hbm,
          src_v, dst_v, msg0, msg1, sem0, sem1, acc_sh):
    _agg_body(g_hbm, src_hbm, dst_hbm, zeros_hbm, out_hbm,
              src_v, dst_v, msg0, msg1, sem0, sem1, acc_sh,
              nch=NCH // 16, core_split_edges=False)


@functools.partial(
    pl.kernel,
    out_type=jax.ShapeDtypeStruct((2, N_PAD, 128), jnp.float32),
    mesh=_mesh,
    scratch_types=[
        pltpu.VMEM((NCH // 32, CH), jnp.int32),
        pltpu.VMEM((NCH // 32, CH), jnp.int32),
        pltpu.VMEM((CH, 128), jnp.float32),
        pltpu.VMEM((CH, 128), jnp.float32),
        pltpu.SemaphoreType.DMA,
        pltpu.SemaphoreType.DMA,
        pltpu.VMEM_SHARED((N_PAD, 128), jnp.float32),
    ],
)
def _agg2(g_hbm, src_hbm, dst_hbm, zeros_hbm, out_hbm,
          src_v, dst_v, msg0, msg1, sem0, sem1, acc_sh):
    _agg_body(g_hbm, src_hbm, dst_hbm, zeros_hbm, out_hbm,
              src_v, dst_v, msg0, msg1, sem0, sem1, acc_sh,
              nch=NCH // 32, core_split_edges=True)


_RB = 512
_NB = N_PAD // _RB


def _mm1_body(deg_ref, x_ref, w_ref, g_ref, dinv_ref):
    deg = deg_ref[0] + deg_ref[1] + 1.0
    dinv = lax.rsqrt(deg)
    dinv_ref[...] = dinv
    g_ref[...] = dinv * jnp.dot(x_ref[...], w_ref[...],
                                preferred_element_type=jnp.float32)


def _mm1(deg2, x_pad, W1):
    return pl.pallas_call(
        _mm1_body,
        grid=(_NB, 2),
        in_specs=[
            pl.BlockSpec((2, _RB, 1), lambda i, c: (0, i, 0)),
            pl.BlockSpec((_RB, IN_DIM), lambda i, c: (i, 0)),
            pl.BlockSpec((IN_DIM, 128), lambda i, c: (0, c)),
        ],
        out_specs=[
            pl.BlockSpec((_RB, 128), lambda i, c: (c * _NB + i, 0)),
            pl.BlockSpec((_RB, 1), lambda i, c: (i, 0)),
        ],
        out_shape=[
            jax.ShapeDtypeStruct((2 * N_PAD, 128), jnp.float32),
            jax.ShapeDtypeStruct((N_PAD, 1), jnp.float32),
        ],
    )(deg2, x_pad, W1)


def _mm2_body(acc_ref, g_ref, dinv_ref, b1_ref, w2_ref, g2_ref):
    dinv = dinv_ref[...]
    h = jnp.concatenate([acc_ref[0] + g_ref[0], acc_ref[1] + g_ref[1]], axis=1)
    z = jnp.maximum(dinv * h + b1_ref[...], 0.0)
    g2_ref[...] = dinv * jnp.dot(z, w2_ref[...],
                                 preferred_element_type=jnp.float32)


def _mm2(acc1, g1r, dinv, b1, W2):
    return pl.pallas_call(
        _mm2_body,
        grid=(_NB,),
        in_specs=[
            pl.BlockSpec((2, _RB, 128), lambda i: (0, i, 0)),
            pl.BlockSpec((2, _RB, 128), lambda i: (0, i, 0)),
            pl.BlockSpec((_RB, 1), lambda i: (i, 0)),
            pl.BlockSpec((1, HID), lambda i: (0, 0)),
            pl.BlockSpec((HID, OUT_DIM), lambda i: (0, 0)),
        ],
        out_specs=pl.BlockSpec((_RB, OUT_DIM), lambda i: (i, 0)),
        out_shape=jax.ShapeDtypeStruct((N_PAD, OUT_DIM), jnp.float32),
    )(acc1, g1r, dinv, b1, W2)


def _final_body(acc_ref, g2_ref, dinv_ref, b2_ref, out_ref):
    out_ref[...] = (dinv_ref[...] * (acc_ref[0] + acc_ref[1] + g2_ref[...])
                    + b2_ref[...])


def _final(acc2, g2, dinv, b2):
    return pl.pallas_call(
        _final_body,
        grid=(_NB,),
        in_specs=[
            pl.BlockSpec((2, _RB, 128), lambda i: (0, i, 0)),
            pl.BlockSpec((_RB, OUT_DIM), lambda i: (i, 0)),
            pl.BlockSpec((_RB, 1), lambda i: (i, 0)),
            pl.BlockSpec((1, OUT_DIM), lambda i: (0, 0)),
        ],
        out_specs=pl.BlockSpec((_RB, OUT_DIM), lambda i: (i, 0)),
        out_shape=jax.ShapeDtypeStruct((N_PAD, OUT_DIM), jnp.float32),
    )(acc2, g2, dinv, b2)


def kernel(x, edge_index, W1, b1, W2, b2):
    src = edge_index[0]
    dst = edge_index[1]
    npad = E_PAD - E
    ar = jnp.arange(npad, dtype=jnp.int32)
    src_p = jnp.concatenate([src, ar % N]).reshape(NCH, CH)
    dst_p = jnp.concatenate([dst, N + ar % (N_PAD - N)]).reshape(NCH, CH)
    src2 = jnp.stack([src_p, src_p + N_PAD])

    x_pad = jnp.pad(x, ((0, N_PAD - N), (0, 0)))
    ones_c = jnp.ones((CH,), jnp.float32)
    zeros_t = jnp.zeros((CH, 128), jnp.float32)

    deg2 = _deg(dst_p, ones_c)
    g1, dinv = _mm1(deg2[..., None], x_pad, W1)
    acc1 = _agg1(g1, src2, dst_p, zeros_t)
    g1r = g1.reshape(2, N_PAD, 128)
    g2 = _mm2(acc1, g1r, dinv, b1.reshape(1, HID), W2)
    acc2 = _agg2(g2, src_p, dst_p, zeros_t)
    out = _final(acc2, g2, dinv, b2.reshape(1, OUT_DIM))
    return out[:N]


# speedup vs baseline: 20.8436x; 20.8436x over previous
import functools

import jax
import jax.numpy as jnp
from jax import lax
from jax.experimental import pallas as pl
from jax.experimental.pallas import tpu as pltpu, tpu_sc as plsc

N = 10000
E = 320000
IN_DIM = 128
HID = 256
OUT_DIM = 128

N_PAD = 10240
E_PAD = 327680
CH = 128
FQ = 64
NCH = E_PAD // CH
N_PT = N_PAD // 16

_mesh = plsc.VectorSubcoreMesh(core_axis_name="c", subcore_axis_name="s")
_sc_params = pltpu.CompilerParams(use_tc_tiling_on_sc=False)


@functools.partial(
    pl.kernel,
    out_type=jax.ShapeDtypeStruct((2, N_PAD), jnp.float32),
    mesh=_mesh,
    scratch_types=[
        pltpu.VMEM((NCH // 32, CH), jnp.int32),
        pltpu.VMEM((CH,), jnp.float32),
        pltpu.VMEM_SHARED((N_PAD,), jnp.float32),
    ],
    compiler_params=_sc_params,
)
def _deg(dst_hbm, ones_hbm, zeros_hbm, out_hbm, dst_v, ones_v, acc_sh):
    c = lax.axis_index("c")
    s = lax.axis_index("s")
    wid = c * 16 + s
    nch = NCH // 32

    pltpu.sync_copy(zeros_hbm, acc_sh.at[pl.ds(s * N_PT, N_PT)])
    pltpu.sync_copy(ones_hbm, ones_v)
    pltpu.sync_copy(dst_hbm.at[pl.ds(wid * nch, nch)], dst_v)
    plsc.subcore_barrier()

    def _body(j, carry):
        pltpu.sync_copy(ones_v, acc_sh.at[dst_v.at[j]], add=True)
        return carry

    lax.fori_loop(0, nch, _body, 0)
    plsc.subcore_barrier()
    pltpu.sync_copy(acc_sh.at[pl.ds(s * N_PT, N_PT)],
                    out_hbm.at[c, pl.ds(s * N_PT, N_PT)])


NBUF = 4


def _agg_body(g_hbm, src_hbm, dst_hbm, zeros_hbm, out_hbm,
              src_v, dst_v, bufs, gsems, ssems, acc_sh,
              *, passes):
    c = lax.axis_index("c")
    s = lax.axis_index("s")
    nch = NCH // 16
    row0 = s * nch

    pltpu.sync_copy(dst_hbm.at[pl.ds(row0, nch)], dst_v)

    def wait_gather(b):
        pltpu.make_async_copy(g_hbm.at[pl.ds(0, CH)], bufs[b], gsems[b]).wait()

    def wait_scatter(b):
        pltpu.make_async_copy(
            bufs[b], acc_sh.at[dst_v.at[0]], ssems[b]).wait()

    for p in range(passes):
        q = c * passes + p
        pltpu.sync_copy(zeros_hbm, acc_sh.at[pl.ds(s * N_PT, N_PT)])
        pltpu.sync_copy(src_hbm.at[q, pl.ds(row0, nch)], src_v)
        plsc.subcore_barrier()

        pltpu.async_copy(g_hbm.at[src_v.at[0]], bufs[0], gsems[0])
        pltpu.async_copy(g_hbm.at[src_v.at[1]], bufs[1], gsems[1])

        @pl.loop(0, nch, step=NBUF)
        def _pipe(jo):
            for b in range(NBUF):
                j = jo + b
                wait_gather(b)
                pltpu.async_copy(bufs[b], acc_sh.at[dst_v.at[j]],
                                 ssems[b], add=True)
                b2 = (b + 2) % NBUF

                @pl.when(j >= 2)
                def _():
                    wait_scatter(b2)

                @pl.when(j + 2 < nch)
                def _():
                    pltpu.async_copy(g_hbm.at[src_v.at[j + 2]],
                                     bufs[b2], gsems[b2])

        wait_scatter((nch - 2) % NBUF)
        wait_scatter((nch - 1) % NBUF)
        plsc.subcore_barrier()
        pltpu.sync_copy(acc_sh.at[pl.ds(s * N_PT, N_PT)],
                        out_hbm.at[q, pl.ds(s * N_PT, N_PT)])


def _make_agg(passes):
    @functools.partial(
        pl.kernel,
        out_type=jax.ShapeDtypeStruct((2 * passes, N_PAD, FQ), jnp.float32),
        mesh=_mesh,
        scratch_types=[
            pltpu.VMEM((NCH // 16, CH), jnp.int32),
            pltpu.VMEM((NCH // 16, CH), jnp.int32),
        ] + [pltpu.VMEM((CH, FQ), jnp.float32)] * NBUF
          + [pltpu.SemaphoreType.DMA] * (2 * NBUF)
          + [pltpu.VMEM_SHARED((N_PAD, FQ), jnp.float32)],
        compiler_params=_sc_params,
    )
    def _agg(g_hbm, src_hbm, dst_hbm, zeros_hbm, out_hbm,
             src_v, dst_v, *rest):
        bufs = rest[:NBUF]
        gsems = rest[NBUF:2 * NBUF]
        ssems = rest[2 * NBUF:3 * NBUF]
        acc_sh = rest[3 * NBUF]
        _agg_body(g_hbm, src_hbm, dst_hbm, zeros_hbm, out_hbm,
                  src_v, dst_v, bufs, gsems, ssems, acc_sh,
                  passes=passes)

    return _agg


_agg1 = _make_agg(2)
_agg2 = _make_agg(1)


_RB = 512
_NB = N_PAD // _RB


def _mm1_body(deg_ref, x_ref, w_ref, g_ref, dinv_ref):
    deg = deg_ref[0] + deg_ref[1] + 1.0
    dinv = lax.rsqrt(deg)
    dinv_ref[...] = dinv
    g = dinv * jnp.dot(x_ref[...], w_ref[...],
                       preferred_element_type=jnp.float32)
    for q in range(4):
        g_ref[q] = g[:, q * FQ:(q + 1) * FQ]


def _mm1(deg2, x_pad, W1):
    return pl.pallas_call(
        _mm1_body,
        grid=(_NB,),
        in_specs=[
            pl.BlockSpec((2, _RB, 1), lambda i: (0, i, 0)),
            pl.BlockSpec((_RB, IN_DIM), lambda i: (i, 0)),
            pl.BlockSpec((IN_DIM, HID), lambda i: (0, 0)),
        ],
        out_specs=[
            pl.BlockSpec((4, _RB, FQ), lambda i: (0, i, 0)),
            pl.BlockSpec((_RB, 1), lambda i: (i, 0)),
        ],
        out_shape=[
            jax.ShapeDtypeStruct((4, N_PAD, FQ), jnp.float32),
            jax.ShapeDtypeStruct((N_PAD, 1), jnp.float32),
        ],
    )(deg2, x_pad, W1)


def _mm2_body(acc_ref, g_ref, dinv_ref, b1_ref, w2_ref, g2_ref):
    dinv = dinv_ref[...]
    h = jnp.concatenate(
        [acc_ref[q] + g_ref[q] for q in range(4)], axis=1)
    z = jnp.maximum(dinv * h + b1_ref[...], 0.0)
    g2 = dinv * jnp.dot(z, w2_ref[...], preferred_element_type=jnp.float32)
    g2_ref[0] = g2[:, :FQ]
    g2_ref[1] = g2[:, FQ:]


def _mm2(acc1, g1r, dinv, b1, W2):
    return pl.pallas_call(
        _mm2_body,
        grid=(_NB,),
        in_specs=[
            pl.BlockSpec((4, _RB, FQ), lambda i: (0, i, 0)),
            pl.BlockSpec((4, _RB, FQ), lambda i: (0, i, 0)),
            pl.BlockSpec((_RB, 1), lambda i: (i, 0)),
            pl.BlockSpec((1, HID), lambda i: (0, 0)),
            pl.BlockSpec((HID, OUT_DIM), lambda i: (0, 0)),
        ],
        out_specs=pl.BlockSpec((2, _RB, FQ), lambda i: (0, i, 0)),
        out_shape=jax.ShapeDtypeStruct((2, N_PAD, FQ), jnp.float32),
    )(acc1, g1r, dinv, b1, W2)


def _final_body(acc_ref, g2_ref, dinv_ref, b2_ref, out_ref):
    h = jnp.concatenate(
        [acc_ref[q] + g2_ref[q] for q in range(2)], axis=1)
    out_ref[...] = dinv_ref[...] * h + b2_ref[...]


def _final(acc2, g2q, dinv, b2):
    return pl.pallas_call(
        _final_body,
        grid=(_NB,),
        in_specs=[
            pl.BlockSpec((2, _RB, FQ), lambda i: (0, i, 0)),
            pl.BlockSpec((2, _RB, FQ), lambda i: (0, i, 0)),
            pl.BlockSpec((_RB, 1), lambda i: (i, 0)),
            pl.BlockSpec((1, OUT_DIM), lambda i: (0, 0)),
        ],
        out_specs=pl.BlockSpec((_RB, OUT_DIM), lambda i: (i, 0)),
        out_shape=jax.ShapeDtypeStruct((N_PAD, OUT_DIM), jnp.float32),
    )(acc2, g2q, dinv, b2)


def kernel(x, edge_index, W1, b1, W2, b2):
    src = edge_index[0]
    dst = edge_index[1]
    npad = E_PAD - E
    ar = jnp.arange(npad, dtype=jnp.int32)
    src_p = jnp.concatenate([src, ar % N]).reshape(NCH, CH)
    dst_p = jnp.concatenate([dst, N + ar % (N_PAD - N)]).reshape(NCH, CH)
    src4 = jnp.stack([src_p + q * N_PAD for q in range(4)])
    src2 = src4[:2]

    x_pad = jnp.pad(x, ((0, N_PAD - N), (0, 0)))
    ones_c = jnp.ones((CH,), jnp.float32)
    zeros_1 = jnp.zeros((N_PT,), jnp.float32)
    zeros_t = jnp.zeros((N_PT, FQ), jnp.float32)

    deg2 = _deg(dst_p, ones_c, zeros_1)
    g1r, dinv = _mm1(deg2[..., None], x_pad, W1)
    g1 = g1r.reshape(4 * N_PAD, FQ)
    acc1 = _agg1(g1, src4, dst_p, zeros_t)
    g2q = _mm2(acc1, g1r, dinv, b1.reshape(1, HID), W2)
    g2 = g2q.reshape(2 * N_PAD, FQ)
    acc2 = _agg2(g2, src2, dst_p, zeros_t)
    out = _final(acc2, g2q, dinv, b2.reshape(1, OUT_DIM))
    return out[:N]


# speedup vs baseline: 24.0414x; 1.1534x over previous
import functools

import jax
import jax.numpy as jnp
from jax import lax
from jax.experimental import pallas as pl
from jax.experimental.pallas import tpu as pltpu, tpu_sc as plsc

N = 10000
E = 320000
IN_DIM = 128
HID = 256
OUT_DIM = 128

N_PAD = 10240
E_PAD = 327680
CH = 128
FQ = 64
NCH = E_PAD // CH
N_PT = N_PAD // 16
NPART = N % CH

_mesh = plsc.VectorSubcoreMesh(core_axis_name="c", subcore_axis_name="s")
_sc_params = pltpu.CompilerParams(use_tc_tiling_on_sc=False,
                                  needs_layout_passes=False)

NBUF = 4


@functools.partial(
    pl.kernel,
    out_type=jax.ShapeDtypeStruct((2, N_PAD), jnp.float32),
    mesh=_mesh,
    scratch_types=[
        pltpu.VMEM((NCH // 32, CH), jnp.int32),
        pltpu.VMEM((CH,), jnp.float32),
        pltpu.VMEM_SHARED((N_PAD,), jnp.float32),
    ],
    compiler_params=_sc_params,
)
def _deg(dst_hbm, ones_hbm, zeros_hbm, out_hbm, dst_v, ones_v, acc_sh):
    c = lax.axis_index("c")
    s = lax.axis_index("s")
    wid = c * 16 + s
    nch = NCH // 32

    pltpu.sync_copy(zeros_hbm, acc_sh.at[pl.ds(s * N_PT, N_PT)])
    pltpu.sync_copy(ones_hbm, ones_v)
    pltpu.sync_copy(dst_hbm.at[pl.ds(wid * nch, nch)], dst_v)
    plsc.subcore_barrier()

    def _body(j, carry):
        pltpu.sync_copy(ones_v, acc_sh.at[dst_v.at[j]], add=True)
        return carry

    lax.fori_loop(0, nch, _body, 0)
    plsc.subcore_barrier()
    pltpu.sync_copy(acc_sh.at[pl.ds(s * N_PT, N_PT)],
                    out_hbm.at[c, pl.ds(s * N_PT, N_PT)])


def _agg_pipeline(g_hbm, src_hbm, dst_v, src_v, bufs, gsems, ssems, acc_sh,
                  *, q, nch, row0):
    pltpu.sync_copy(src_hbm.at[q, pl.ds(row0, nch)], src_v)
    plsc.subcore_barrier()

    def wait_gather(b):
        pltpu.make_async_copy(g_hbm.at[pl.ds(0, CH)], bufs[b], gsems[b]).wait()

    def wait_scatter(b):
        pltpu.make_async_copy(bufs[b], acc_sh.at[dst_v.at[0]],
                              ssems[b]).wait()

    pltpu.async_copy(g_hbm.at[src_v.at[0]], bufs[0], gsems[0])
    pltpu.async_copy(g_hbm.at[src_v.at[1]], bufs[1], gsems[1])

    @pl.loop(0, nch, step=NBUF)
    def _pipe(jo):
        for b in range(NBUF):
            j = jo + b
            wait_gather(b)
            pltpu.async_copy(bufs[b], acc_sh.at[dst_v.at[j]],
                             ssems[b], add=True)
            b2 = (b + 2) % NBUF

            @pl.when(j >= 2)
            def _():
                wait_scatter(b2)

            @pl.when(j + 2 < nch)
            def _():
                pltpu.async_copy(g_hbm.at[src_v.at[j + 2]],
                                 bufs[b2], gsems[b2])

    wait_scatter((nch - 2) % NBUF)
    wait_scatter((nch - 1) % NBUF)
    plsc.subcore_barrier()


@functools.partial(
    pl.kernel,
    out_type=jax.ShapeDtypeStruct((2, N_PAD, 128), jnp.float32),
    mesh=_mesh,
    scratch_types=[
        pltpu.VMEM((NCH // 16, CH), jnp.int32),
        pltpu.VMEM((NCH // 16, CH), jnp.int32),
    ] + [pltpu.VMEM((CH, FQ), jnp.float32)] * NBUF
      + [pltpu.SemaphoreType.DMA] * (2 * NBUF)
      + [pltpu.VMEM_SHARED((N_PAD, FQ), jnp.float32)],
    compiler_params=_sc_params,
)
def _agg1(g_hbm, src_hbm, dst_hbm, zeros_hbm, out_hbm, src_v, dst_v, *rest):
    bufs = rest[:NBUF]
    gsems = rest[NBUF:2 * NBUF]
    ssems = rest[2 * NBUF:3 * NBUF]
    acc_sh = rest[3 * NBUF]
    c = lax.axis_index("c")
    s = lax.axis_index("s")
    nch = NCH // 16
    row0 = s * nch

    pltpu.sync_copy(dst_hbm.at[pl.ds(row0, nch)], dst_v)
    for p in range(2):
        pltpu.sync_copy(zeros_hbm, acc_sh.at[pl.ds(s * N_PT, N_PT)])
        _agg_pipeline(g_hbm, src_hbm, dst_v, src_v, bufs, gsems, ssems,
                      acc_sh, q=c * 2 + p, nch=nch, row0=row0)
        pltpu.sync_copy(acc_sh.at[pl.ds(s * N_PT, N_PT)],
                        out_hbm.at[c, pl.ds(s * N_PT, N_PT),
                                   pl.ds(p * FQ, FQ)])
        if p == 0:
            plsc.subcore_barrier()


@functools.partial(
    pl.kernel,
    out_type=jax.ShapeDtypeStruct((N, 128), jnp.float32),
    mesh=_mesh,
    scratch_types=[
        pltpu.VMEM((NCH // 16, CH), jnp.int32),
        pltpu.VMEM((NCH // 16, CH), jnp.int32),
    ] + [pltpu.VMEM((CH, FQ), jnp.float32)] * NBUF
      + [pltpu.SemaphoreType.DMA] * (2 * NBUF)
      + [pltpu.VMEM((CH, FQ), jnp.float32),
         pltpu.VMEM((CH,), jnp.int32),
         pltpu.VMEM((CH,), jnp.float32),
         pltpu.VMEM((FQ,), jnp.float32),
         pltpu.VMEM_SHARED((N_PAD, FQ), jnp.float32)],
    compiler_params=_sc_params,
)
def _agg2(g_hbm, src_hbm, dst_hbm, zeros_hbm, dinv_hbm, b2_hbm,
          out_hbm, src_v, dst_v, *rest):
    bufs = rest[:NBUF]
    gsems = rest[NBUF:2 * NBUF]
    ssems = rest[2 * NBUF:3 * NBUF]
    g2_v, gi_v, dv_v, b2_v, acc_sh = rest[3 * NBUF:]
    c = lax.axis_index("c")
    s = lax.axis_index("s")
    nch = NCH // 16
    row0 = s * nch

    pltpu.sync_copy(dst_hbm.at[pl.ds(row0, nch)], dst_v)
    pltpu.sync_copy(zeros_hbm, acc_sh.at[pl.ds(s * N_PT, N_PT)])
    pltpu.sync_copy(b2_hbm.at[c], b2_v)
    _agg_pipeline(g_hbm, src_hbm, dst_v, src_v, bufs, gsems, ssems,
                  acc_sh, q=c, nch=nch, row0=row0)

    b2k = [b2_v[pl.ds(k * 16, 16)] for k in range(4)]
    for kb in range(N_PT // CH):
        r0 = s * N_PT + kb * CH
        pltpu.sync_copy(acc_sh.at[pl.ds(r0, CH)], bufs[0])
        for k in range(CH // 16):
            gi_v[pl.ds(k * 16, 16)] = (
                2 * lax.iota(jnp.int32, 16) + (2 * (r0 + k * 16) + c))
        pltpu.sync_copy(g_hbm.at[gi_v], g2_v)
        pltpu.sync_copy(dinv_hbm.at[pl.ds(r0, CH)], dv_v)

        def _row(r, carry):
            dv = plsc.load_gather(dv_v, [jnp.full((16,), r, jnp.int32)])
            for k in range(4):
                sl = pl.ds(k * 16, 16)
                a = bufs[0][r, sl]
                g = g2_v[r, sl]
                bufs[1][r, sl] = (a + g) * dv + b2k[k]
            return carry

        lax.fori_loop(0, CH, _row, 0)

        @pl.when(r0 + CH <= N)
        def _():
            pltpu.sync_copy(bufs[1],
                            out_hbm.at[pl.ds(r0, CH), pl.ds(c * FQ, FQ)])

        @pl.when((r0 < N) & (r0 + CH > N))
        def _():
            pltpu.sync_copy(bufs[1].at[pl.ds(0, NPART)],
                            out_hbm.at[pl.ds(r0, NPART), pl.ds(c * FQ, FQ)])


_RB = 512
_NB = N_PAD // _RB


def _mm1_body(deg_ref, x_ref, w_ref, g_ref, dinv_ref):
    deg = deg_ref[0] + deg_ref[1] + 1.0
    dinv = lax.rsqrt(deg)[:, None]
    dinv_ref[...] = dinv
    g = dinv * jnp.dot(x_ref[...], w_ref[...],
                       preferred_element_type=jnp.float32)
    g_ref[0] = g[:, :128]
    g_ref[1] = g[:, 128:]


def _mm1(deg2, x_pad, W1):
    return pl.pallas_call(
        _mm1_body,
        grid=(_NB,),
        in_specs=[
            pl.BlockSpec((2, _RB), lambda i: (0, i)),
            pl.BlockSpec((_RB, IN_DIM), lambda i: (i, 0)),
            pl.BlockSpec((IN_DIM, HID), lambda i: (0, 0)),
        ],
        out_specs=[
            pl.BlockSpec((2, _RB, 128), lambda i: (0, i, 0)),
            pl.BlockSpec((_RB, 1), lambda i: (i, 0)),
        ],
        out_shape=[
            jax.ShapeDtypeStruct((2, N_PAD, 128), jnp.float32),
            jax.ShapeDtypeStruct((N_PAD, 1), jnp.float32),
        ],
    )(deg2, x_pad, W1)


def _mm2_body(acc_ref, g_ref, dinv_ref, b1_ref, w2_ref, g2_ref):
    dinv = dinv_ref[...]
    h = jnp.concatenate([acc_ref[0] + g_ref[0], acc_ref[1] + g_ref[1]],
                        axis=1)
    z = jnp.maximum(dinv * h + b1_ref[...], 0.0)
    g2_ref[...] = dinv * jnp.dot(z, w2_ref[...],
                                 preferred_element_type=jnp.float32)


def _mm2(acc1, g1h, dinv, b1, W2):
    return pl.pallas_call(
        _mm2_body,
        grid=(_NB,),
        in_specs=[
            pl.BlockSpec((2, _RB, 128), lambda i: (0, i, 0)),
            pl.BlockSpec((2, _RB, 128), lambda i: (0, i, 0)),
            pl.BlockSpec((_RB, 1), lambda i: (i, 0)),
            pl.BlockSpec((1, HID), lambda i: (0, 0)),
            pl.BlockSpec((HID, OUT_DIM), lambda i: (0, 0)),
        ],
        out_specs=pl.BlockSpec((_RB, OUT_DIM), lambda i: (i, 0)),
        out_shape=jax.ShapeDtypeStruct((N_PAD, OUT_DIM), jnp.float32),
    )(acc1, g1h, dinv, b1, W2)


def kernel(x, edge_index, W1, b1, W2, b2):
    src = edge_index[0]
    dst = edge_index[1]
    npad = E_PAD - E
    ar = jnp.arange(npad, dtype=jnp.int32)
    src_p = jnp.concatenate([src, ar % N]).reshape(NCH, CH)
    dst_p = jnp.concatenate([dst, N + ar % (N_PAD - N)]).reshape(NCH, CH)
    src4 = jnp.stack([2 * src_p + 2 * (q // 2) * N_PAD + (q % 2)
                      for q in range(4)])
    src2 = jnp.stack([2 * src_p, 2 * src_p + 1])

    x_pad = jnp.pad(x, ((0, N_PAD - N), (0, 0)))
    ones_c = jnp.ones((CH,), jnp.float32)
    zeros_1 = jnp.zeros((N_PT,), jnp.float32)
    zeros_t = jnp.zeros((N_PT, FQ), jnp.float32)

    deg2 = _deg(dst_p, ones_c, zeros_1)
    g1h, dinv = _mm1(deg2, x_pad, W1)
    g1_64 = g1h.reshape(4 * N_PAD, FQ)
    acc1 = _agg1(g1_64, src4, dst_p, zeros_t)
    g2 = _mm2(acc1, g1h, dinv, b1.reshape(1, HID), W2)
    g2_64 = g2.reshape(2 * N_PAD, FQ)
    out = _agg2(g2_64, src2, dst_p, zeros_t,
                dinv.reshape(N_PAD), b2.reshape(2, FQ))
    return out
